# baseline (speedup 1.0000x reference)
import functools

import jax
import jax.numpy as jnp
from jax.experimental import pallas as pl
from jax.experimental.pallas import tpu as pltpu


def _qkv_kernel(hs_ref, wq_ref, wk_ref, wv_ref, bq_ref, bk_ref, bv_ref,
                q_ref, k_ref, v_ref, *, hp, d):
    f32 = jnp.float32
    bf16 = jnp.bfloat16
    a = hs_ref[...].astype(bf16)
    def proj(w_ref, b_ref):
        acc = jnp.dot(a, w_ref[...], preferred_element_type=f32)
        return (acc + b_ref[...]).astype(bf16)
    q = proj(wq_ref, bq_ref)
    k = proj(wk_ref, bk_ref)
    v = proj(wv_ref, bv_ref)
    for hh in range(hp):
        sl = slice(hh * d, (hh + 1) * d)
        q_ref[hh, 0] = q[:, sl]
        k_ref[hh, 0] = k[:, sl]
        v_ref[hh, 0] = v[:, sl]


def _z_kernel(xq_ref, xk_ref, wp_ref, zq_ref, zk_ref):
    wp = wp_ref[...]
    f32 = jnp.float32
    bf16 = jnp.bfloat16
    zq_ref[...] = jnp.dot(xq_ref[...], wp, preferred_element_type=f32).astype(bf16)
    zk_ref[...] = jnp.dot(xk_ref[...], wp, preferred_element_type=f32).astype(bf16)


def _attn_kernel(zq_ref, zk_ref, v_ref, o_ref, *, inv_t):
    zq = zq_ref[0]
    zk = zk_ref[0]
    one = jnp.bfloat16(1.0)
    zero = jnp.bfloat16(0.0)
    w = jnp.maximum(one - jnp.abs(zq - zk), zero)
    o_ref[0] = jnp.dot(w, v_ref[0], preferred_element_type=jnp.float32) * inv_t


def kernel(hidden_states, Wq, bq, Wk, bk, Wv, bv, Wp, log_bandwidth):
    f32 = jnp.float32
    Bs, T, Hid = hidden_states.shape
    H = Wp.shape[0]
    D = Hid // H
    HB = H * Bs
    M = Bs * T
    R = T // H

    bf16 = jnp.bfloat16
    hs2 = hidden_states.reshape(M, Hid)
    WqT = Wq.T.astype(bf16)
    WkT = Wk.T.astype(bf16)
    WvT = Wv.T.astype(bf16)
    bq2 = bq[None, :].astype(f32)
    bk2 = bk[None, :].astype(f32)
    bv2 = bv[None, :].astype(f32)

    TM = min(256, T)
    n_t = T // TM
    HP = min(16, H)
    w_spec = pl.BlockSpec((Hid, HP * D), lambda h, i: (0, h))
    b_spec = pl.BlockSpec((1, HP * D), lambda h, i: (0, h))
    o_spec = pl.BlockSpec((HP, 1, TM, D),
                          lambda h, i: (h, i // n_t, i % n_t, 0))
    qh, kh, vh = pl.pallas_call(
        functools.partial(_qkv_kernel, hp=HP, d=D),
        grid=(H // HP, M // TM),
        in_specs=[
            pl.BlockSpec((TM, Hid), lambda h, i: (i, 0)),
            w_spec, w_spec, w_spec, b_spec, b_spec, b_spec,
        ],
        out_specs=[o_spec, o_spec, o_spec],
        out_shape=[jax.ShapeDtypeStruct((H, Bs, T, D), bf16)] * 3,
        compiler_params=pltpu.CompilerParams(
            dimension_semantics=("parallel", "parallel")),
    )(hs2, WqT, WkT, WvT, bq2, bk2, bv2)

    xq = qh.reshape(HB * R, Hid)
    xk = kh.reshape(HB * R, Hid)

    bw = jax.nn.softplus(log_bandwidth.astype(f32)) + 1e-4
    Wp_s = (Wp.astype(f32) / bw[:, None]).T
    NP = 128
    Wp_pad = jnp.zeros((Hid, NP), f32).at[:, :H].set(Wp_s).astype(bf16)

    TMZ = min(512, HB * R)
    zq_r, zk_r = pl.pallas_call(
        _z_kernel,
        grid=(HB * R // TMZ,),
        in_specs=[
            pl.BlockSpec((TMZ, Hid), lambda i: (i, 0)),
            pl.BlockSpec((TMZ, Hid), lambda i: (i, 0)),
            pl.BlockSpec((Hid, NP), lambda i: (0, 0)),
        ],
        out_specs=[
            pl.BlockSpec((TMZ, NP), lambda i: (i, 0)),
            pl.BlockSpec((TMZ, NP), lambda i: (i, 0)),
        ],
        out_shape=[jax.ShapeDtypeStruct((HB * R, NP), bf16)] * 2,
        compiler_params=pltpu.CompilerParams(
            dimension_semantics=("parallel",)),
    )(xq, xk, Wp_pad)

    def to_hb(z_r):
        z4 = z_r.reshape(H, Bs, R, NP)[..., :H]
        return z4.transpose(3, 1, 0, 2).reshape(HB, T)

    zq = to_hb(zq_r).reshape(HB, T, 1)
    zk = to_hb(zk_r).reshape(HB, 1, T)
    v3 = vh.reshape(HB, T, D)

    TQ = min(1024, T)
    out = pl.pallas_call(
        functools.partial(_attn_kernel, inv_t=1.0 / T),
        grid=(HB, T // TQ),
        in_specs=[
            pl.BlockSpec((1, TQ, 1), lambda g, i: (g, i, 0)),
            pl.BlockSpec((1, 1, T), lambda g, i: (g, 0, 0)),
            pl.BlockSpec((1, T, D), lambda g, i: (g, 0, 0)),
        ],
        out_specs=pl.BlockSpec((1, TQ, D),
                               lambda g, i: (g % Bs, i, g // Bs)),
        out_shape=jax.ShapeDtypeStruct((Bs, T, Hid), f32),
        compiler_params=pltpu.CompilerParams(
            dimension_semantics=("parallel", "parallel")),
    )(zq, zk, v3)

    return out


# speedup vs baseline: 1.0841x; 1.0841x over previous
import functools

import jax
import jax.numpy as jnp
from jax.experimental import pallas as pl
from jax.experimental.pallas import tpu as pltpu


def _qkv_kernel(hs_ref, wq_ref, wk_ref, wv_ref, bq_ref, bk_ref, bv_ref,
                q_ref, k_ref, v_ref, *, hp, d):
    f32 = jnp.float32
    bf16 = jnp.bfloat16
    a = hs_ref[...].astype(bf16)
    dn = (((1,), (1,)), ((), ()))
    def proj(w_ref, b_ref):
        w = w_ref[...].astype(bf16)
        acc = jax.lax.dot_general(a, w, dn, preferred_element_type=f32)
        return (acc + b_ref[...]).astype(bf16)
    q = proj(wq_ref, bq_ref)
    k = proj(wk_ref, bk_ref)
    v = proj(wv_ref, bv_ref)
    for hh in range(hp):
        sl = slice(hh * d, (hh + 1) * d)
        q_ref[hh, 0] = q[:, sl]
        k_ref[hh, 0] = k[:, sl]
        v_ref[hh, 0] = v[:, sl]


def _z_kernel(xq_ref, xk_ref, wp_ref, zq_ref, zk_ref):
    wp = wp_ref[...]
    f32 = jnp.float32
    bf16 = jnp.bfloat16
    zq_ref[...] = jnp.dot(xq_ref[...], wp, preferred_element_type=f32).astype(bf16)
    zk_ref[...] = jnp.dot(xk_ref[...], wp, preferred_element_type=f32).astype(bf16)


def _attn_kernel(zq_ref, zk_ref, v_ref, o_ref, *, inv_t):
    zq = zq_ref[0]
    zk = zk_ref[0]
    one = jnp.bfloat16(1.0)
    zero = jnp.bfloat16(0.0)
    w = jnp.maximum(one - jnp.abs(zq - zk), zero)
    o_ref[0] = jnp.dot(w, v_ref[0], preferred_element_type=jnp.float32) * inv_t


def kernel(hidden_states, Wq, bq, Wk, bk, Wv, bv, Wp, log_bandwidth):
    f32 = jnp.float32
    Bs, T, Hid = hidden_states.shape
    H = Wp.shape[0]
    D = Hid // H
    HB = H * Bs
    M = Bs * T
    R = T // H

    bf16 = jnp.bfloat16
    hs2 = hidden_states.reshape(M, Hid)
    WqT = Wq
    WkT = Wk
    WvT = Wv
    bq2 = bq[None, :].astype(f32)
    bk2 = bk[None, :].astype(f32)
    bv2 = bv[None, :].astype(f32)

    TM = min(256, T)
    n_t = T // TM
    HP = min(8, H)
    w_spec = pl.BlockSpec((HP * D, Hid), lambda h, i: (h, 0))
    b_spec = pl.BlockSpec((1, HP * D), lambda h, i: (0, h))
    o_spec = pl.BlockSpec((HP, 1, TM, D),
                          lambda h, i: (h, i // n_t, i % n_t, 0))
    qh, kh, vh = pl.pallas_call(
        functools.partial(_qkv_kernel, hp=HP, d=D),
        grid=(H // HP, M // TM),
        in_specs=[
            pl.BlockSpec((TM, Hid), lambda h, i: (i, 0)),
            w_spec, w_spec, w_spec, b_spec, b_spec, b_spec,
        ],
        out_specs=[o_spec, o_spec, o_spec],
        out_shape=[jax.ShapeDtypeStruct((H, Bs, T, D), bf16)] * 3,
        compiler_params=pltpu.CompilerParams(
            dimension_semantics=("parallel", "parallel")),
    )(hs2, WqT, WkT, WvT, bq2, bk2, bv2)

    xq = qh.reshape(HB * R, Hid)
    xk = kh.reshape(HB * R, Hid)

    bw = jax.nn.softplus(log_bandwidth.astype(f32)) + 1e-4
    Wp_s = (Wp.astype(f32) / bw[:, None]).T
    NP = 128
    Wp_pad = jnp.zeros((Hid, NP), f32).at[:, :H].set(Wp_s).astype(bf16)

    TMZ = min(512, HB * R)
    zq_r, zk_r = pl.pallas_call(
        _z_kernel,
        grid=(HB * R // TMZ,),
        in_specs=[
            pl.BlockSpec((TMZ, Hid), lambda i: (i, 0)),
            pl.BlockSpec((TMZ, Hid), lambda i: (i, 0)),
            pl.BlockSpec((Hid, NP), lambda i: (0, 0)),
        ],
        out_specs=[
            pl.BlockSpec((TMZ, NP), lambda i: (i, 0)),
            pl.BlockSpec((TMZ, NP), lambda i: (i, 0)),
        ],
        out_shape=[jax.ShapeDtypeStruct((HB * R, NP), bf16)] * 2,
        compiler_params=pltpu.CompilerParams(
            dimension_semantics=("parallel",)),
    )(xq, xk, Wp_pad)

    def to_hb(z_r):
        z4 = z_r.reshape(H, Bs, R, NP)[..., :H]
        return z4.transpose(3, 1, 0, 2).reshape(HB, T)

    zq = to_hb(zq_r).reshape(HB, T, 1)
    zk = to_hb(zk_r).reshape(HB, 1, T)
    v3 = vh.reshape(HB, T, D)

    TQ = min(1024, T)
    out = pl.pallas_call(
        functools.partial(_attn_kernel, inv_t=1.0 / T),
        grid=(HB, T // TQ),
        in_specs=[
            pl.BlockSpec((1, TQ, 1), lambda g, i: (g, i, 0)),
            pl.BlockSpec((1, 1, T), lambda g, i: (g, 0, 0)),
            pl.BlockSpec((1, T, D), lambda g, i: (g, 0, 0)),
        ],
        out_specs=pl.BlockSpec((1, TQ, D),
                               lambda g, i: (g % Bs, i, g // Bs)),
        out_shape=jax.ShapeDtypeStruct((Bs, T, Hid), f32),
        compiler_params=pltpu.CompilerParams(
            dimension_semantics=("parallel", "parallel")),
    )(zq, zk, v3)

    return out


# speedup vs baseline: 1.1183x; 1.0315x over previous
import functools

import jax
import jax.numpy as jnp
from jax.experimental import pallas as pl
from jax.experimental.pallas import tpu as pltpu


def _qkv_kernel(hs_ref, wq_ref, wk_ref, wv_ref, bq_ref, bk_ref, bv_ref,
                q_ref, k_ref, v_ref, *, hp, d):
    f32 = jnp.float32
    bf16 = jnp.bfloat16
    a = hs_ref[...].astype(bf16)
    dn = (((1,), (1,)), ((), ()))
    def proj(w_ref, b_ref):
        w = w_ref[...].astype(bf16)
        acc = jax.lax.dot_general(a, w, dn, preferred_element_type=f32)
        return (acc + b_ref[...]).astype(bf16)
    q = proj(wq_ref, bq_ref)
    k = proj(wk_ref, bk_ref)
    v = proj(wv_ref, bv_ref)
    for hh in range(hp):
        sl = slice(hh * d, (hh + 1) * d)
        q_ref[hh, 0] = q[:, sl]
        k_ref[hh, 0] = k[:, sl]
        v_ref[hh, 0] = v[:, sl]


def _z_kernel(xq_ref, xk_ref, wp_ref, zq_ref, zk_ref):
    wp = wp_ref[...]
    f32 = jnp.float32
    bf16 = jnp.bfloat16
    zq_ref[...] = jnp.dot(xq_ref[...], wp, preferred_element_type=f32).astype(bf16)
    zk_ref[...] = jnp.dot(xk_ref[...], wp, preferred_element_type=f32).astype(bf16)


def _attn_kernel(zq_ref, zk_ref, v_ref, o_ref, *, inv_t):
    zq = zq_ref[0]
    zk = zk_ref[0]
    one = jnp.bfloat16(1.0)
    zero = jnp.bfloat16(0.0)
    w = jnp.maximum(one - jnp.abs(zq - zk), zero)
    o_ref[0] = jnp.dot(w, v_ref[0], preferred_element_type=jnp.float32) * inv_t


def kernel(hidden_states, Wq, bq, Wk, bk, Wv, bv, Wp, log_bandwidth):
    f32 = jnp.float32
    Bs, T, Hid = hidden_states.shape
    H = Wp.shape[0]
    D = Hid // H
    HB = H * Bs
    M = Bs * T
    R = T // H

    bf16 = jnp.bfloat16
    hs2 = hidden_states.reshape(M, Hid)
    WqT = Wq
    WkT = Wk
    WvT = Wv
    bq2 = bq[None, :].astype(f32)
    bk2 = bk[None, :].astype(f32)
    bv2 = bv[None, :].astype(f32)

    TM = min(256, T)
    n_t = T // TM
    HP = min(8, H)
    w_spec = pl.BlockSpec((HP * D, Hid), lambda h, i: (h, 0))
    b_spec = pl.BlockSpec((1, HP * D), lambda h, i: (0, h))
    o_spec = pl.BlockSpec((HP, 1, TM, D),
                          lambda h, i: (h, i // n_t, i % n_t, 0))
    qh, kh, vh = pl.pallas_call(
        functools.partial(_qkv_kernel, hp=HP, d=D),
        grid=(H // HP, M // TM),
        in_specs=[
            pl.BlockSpec((TM, Hid), lambda h, i: (i, 0)),
            w_spec, w_spec, w_spec, b_spec, b_spec, b_spec,
        ],
        out_specs=[o_spec, o_spec, o_spec],
        out_shape=[jax.ShapeDtypeStruct((H, Bs, T, D), bf16)] * 3,
        compiler_params=pltpu.CompilerParams(
            dimension_semantics=("parallel", "parallel")),
    )(hs2, WqT, WkT, WvT, bq2, bk2, bv2)

    xq = qh.reshape(HB * R, Hid)
    xk = kh.reshape(HB * R, Hid)

    bw = jax.nn.softplus(log_bandwidth.astype(f32)) + 1e-4
    Wp_s = (Wp.astype(f32) / bw[:, None]).T
    NP = 128
    Wp_pad = jnp.zeros((Hid, NP), f32).at[:, :H].set(Wp_s).astype(bf16)

    TMZ = min(512, HB * R)
    zq_r, zk_r = pl.pallas_call(
        _z_kernel,
        grid=(HB * R // TMZ,),
        in_specs=[
            pl.BlockSpec((TMZ, Hid), lambda i: (i, 0)),
            pl.BlockSpec((TMZ, Hid), lambda i: (i, 0)),
            pl.BlockSpec((Hid, NP), lambda i: (0, 0)),
        ],
        out_specs=[
            pl.BlockSpec((TMZ, NP), lambda i: (i, 0)),
            pl.BlockSpec((TMZ, NP), lambda i: (i, 0)),
        ],
        out_shape=[jax.ShapeDtypeStruct((HB * R, NP), bf16)] * 2,
        compiler_params=pltpu.CompilerParams(
            dimension_semantics=("parallel",)),
    )(xq, xk, Wp_pad)

    def to_hb(z_r):
        z4 = z_r.reshape(H, Bs, R, NP)[..., :H]
        return z4.transpose(3, 1, 0, 2).reshape(HB, T)

    zq = to_hb(zq_r).reshape(HB, T, 1)
    zk = to_hb(zk_r).reshape(HB, 1, T)
    v3 = vh.reshape(HB, T, D)

    TQ = min(2048, T)
    out = pl.pallas_call(
        functools.partial(_attn_kernel, inv_t=1.0 / T),
        grid=(HB, T // TQ),
        in_specs=[
            pl.BlockSpec((1, TQ, 1), lambda g, i: (g, i, 0)),
            pl.BlockSpec((1, 1, T), lambda g, i: (g, 0, 0)),
            pl.BlockSpec((1, T, D), lambda g, i: (g, 0, 0)),
        ],
        out_specs=pl.BlockSpec((1, TQ, D),
                               lambda g, i: (g % Bs, i, g // Bs)),
        out_shape=jax.ShapeDtypeStruct((Bs, T, Hid), f32),
        compiler_params=pltpu.CompilerParams(
            dimension_semantics=("parallel", "parallel")),
    )(zq, zk, v3)

    return out


# speedup vs baseline: 1.1481x; 1.0266x over previous
import functools

import jax
import jax.numpy as jnp
from jax.experimental import pallas as pl
from jax.experimental.pallas import tpu as pltpu


def _qkv_kernel(hs_ref, wq_ref, wk_ref, wv_ref, bq_ref, bk_ref, bv_ref,
                q_ref, k_ref, v_ref, *, hp, d):
    f32 = jnp.float32
    bf16 = jnp.bfloat16
    a = hs_ref[...].astype(bf16)
    dn = (((1,), (1,)), ((), ()))
    def proj(w_ref, b_ref):
        w = w_ref[...].astype(bf16)
        acc = jax.lax.dot_general(a, w, dn, preferred_element_type=f32)
        return (acc + b_ref[...]).astype(bf16)
    q = proj(wq_ref, bq_ref)
    k = proj(wk_ref, bk_ref)
    v = proj(wv_ref, bv_ref)
    for hh in range(hp):
        sl = slice(hh * d, (hh + 1) * d)
        q_ref[hh, 0] = q[:, sl]
        k_ref[hh, 0] = k[:, sl]
        v_ref[hh, 0] = v[:, sl]


def _z_kernel(xq_ref, xk_ref, wp_ref, zq_ref, zk_ref):
    wp = wp_ref[...]
    f32 = jnp.float32
    bf16 = jnp.bfloat16
    zq_ref[...] = jnp.dot(xq_ref[...], wp, preferred_element_type=f32).astype(bf16)
    zk_ref[...] = jnp.dot(xk_ref[...], wp, preferred_element_type=f32).astype(bf16)


def _attn_kernel(zq_ref, zk_ref, v_ref, o_ref, *, inv_t):
    zq = zq_ref[0]
    zk = zk_ref[0]
    one = jnp.bfloat16(1.0)
    zero = jnp.bfloat16(0.0)
    w = jnp.maximum(one - jnp.abs(zq - zk), zero)
    o_ref[0] = jnp.dot(w, v_ref[0], preferred_element_type=jnp.float32) * inv_t


def kernel(hidden_states, Wq, bq, Wk, bk, Wv, bv, Wp, log_bandwidth):
    f32 = jnp.float32
    Bs, T, Hid = hidden_states.shape
    H = Wp.shape[0]
    D = Hid // H
    HB = H * Bs
    M = Bs * T
    R = T // H

    bf16 = jnp.bfloat16
    hs2 = hidden_states.reshape(M, Hid)
    WqT = Wq
    WkT = Wk
    WvT = Wv
    bq2 = bq[None, :].astype(f32)
    bk2 = bk[None, :].astype(f32)
    bv2 = bv[None, :].astype(f32)

    TM = min(512, T)
    n_t = T // TM
    HP = min(8, H)
    w_spec = pl.BlockSpec((HP * D, Hid), lambda h, i: (h, 0))
    b_spec = pl.BlockSpec((1, HP * D), lambda h, i: (0, h))
    o_spec = pl.BlockSpec((HP, 1, TM, D),
                          lambda h, i: (h, i // n_t, i % n_t, 0))
    qh, kh, vh = pl.pallas_call(
        functools.partial(_qkv_kernel, hp=HP, d=D),
        grid=(H // HP, M // TM),
        in_specs=[
            pl.BlockSpec((TM, Hid), lambda h, i: (i, 0)),
            w_spec, w_spec, w_spec, b_spec, b_spec, b_spec,
        ],
        out_specs=[o_spec, o_spec, o_spec],
        out_shape=[jax.ShapeDtypeStruct((H, Bs, T, D), bf16)] * 3,
        compiler_params=pltpu.CompilerParams(
            dimension_semantics=("parallel", "parallel"),
            vmem_limit_bytes=64 * 1024 * 1024),
    )(hs2, WqT, WkT, WvT, bq2, bk2, bv2)

    xq = qh.reshape(HB * R, Hid)
    xk = kh.reshape(HB * R, Hid)

    bw = jax.nn.softplus(log_bandwidth.astype(f32)) + 1e-4
    Wp_s = (Wp.astype(f32) / bw[:, None]).T
    NP = 128
    Wp_pad = jnp.zeros((Hid, NP), f32).at[:, :H].set(Wp_s).astype(bf16)

    TMZ = min(2048, HB * R)
    zq_r, zk_r = pl.pallas_call(
        _z_kernel,
        grid=(HB * R // TMZ,),
        in_specs=[
            pl.BlockSpec((TMZ, Hid), lambda i: (i, 0)),
            pl.BlockSpec((TMZ, Hid), lambda i: (i, 0)),
            pl.BlockSpec((Hid, NP), lambda i: (0, 0)),
        ],
        out_specs=[
            pl.BlockSpec((TMZ, NP), lambda i: (i, 0)),
            pl.BlockSpec((TMZ, NP), lambda i: (i, 0)),
        ],
        out_shape=[jax.ShapeDtypeStruct((HB * R, NP), bf16)] * 2,
        compiler_params=pltpu.CompilerParams(
            dimension_semantics=("parallel",)),
    )(xq, xk, Wp_pad)

    def to_hb(z_r):
        z4 = z_r.reshape(H, Bs, R, NP)[..., :H]
        return z4.transpose(3, 1, 0, 2).reshape(HB, T)

    zq = to_hb(zq_r).reshape(HB, T, 1)
    zk = to_hb(zk_r).reshape(HB, 1, T)
    v3 = vh.reshape(HB, T, D)

    TQ = min(2048, T)
    out = pl.pallas_call(
        functools.partial(_attn_kernel, inv_t=1.0 / T),
        grid=(HB, T // TQ),
        in_specs=[
            pl.BlockSpec((1, TQ, 1), lambda g, i: (g, i, 0)),
            pl.BlockSpec((1, 1, T), lambda g, i: (g, 0, 0)),
            pl.BlockSpec((1, T, D), lambda g, i: (g, 0, 0)),
        ],
        out_specs=pl.BlockSpec((1, TQ, D),
                               lambda g, i: (g % Bs, i, g // Bs)),
        out_shape=jax.ShapeDtypeStruct((Bs, T, Hid), f32),
        compiler_params=pltpu.CompilerParams(
            dimension_semantics=("parallel", "parallel")),
    )(zq, zk, v3)

    return out
